# baseline (speedup 1.0000x reference)
import dataclasses
import functools

import jax
import jax.numpy as jnp
from jax import lax
from jax.experimental import pallas as pl
from jax.experimental.pallas import tpu as pltpu
from jax.experimental.pallas import tpu_sc as plsc

K = 100000
D = 128

S = 98560
KSC = K - S

NC = 2
NS = 16
NW = NC * NS
RPW = KSC // NW
NBUF = 1
CH = 45
NROUND = RPW // (NBUF * CH)
assert RPW == NBUF * CH * NROUND
U = 5
NV = D // 16

BT = 1280
NT = S // BT
assert S == BT * NT
TPB = BT // 128


def _sc_l1_argmin(keys_flat, query_flat):
    mesh = plsc.VectorSubcoreMesh(core_axis_name="c", subcore_axis_name="s")
    cp = pltpu.CompilerParams()
    if "needs_layout_passes" in pltpu.CompilerParams.__dataclass_fields__:
        cp = dataclasses.replace(cp, needs_layout_passes=False)

    @functools.partial(
        pl.kernel,
        mesh=mesh,
        compiler_params=cp,
        out_type=[
            jax.ShapeDtypeStruct((NW, 16), jnp.float32),
            jax.ShapeDtypeStruct((NW, 16), jnp.int32),
        ],
        scratch_types=[pltpu.VMEM((CH * D,), jnp.float32)] * NBUF + [
            pltpu.VMEM((D,), jnp.float32),
            pltpu.VMEM((16,), jnp.float32),
            pltpu.VMEM((16,), jnp.int32),
        ] + [pltpu.SemaphoreType.DMA] * NBUF,
    )
    def k(keys_hbm, q_hbm, out_v_hbm, out_i_hbm, *rest):
        bufs = rest[:NBUF]
        q_v, res_v, resi_v = rest[NBUF:NBUF + 3]
        sems = rest[NBUF + 3:]
        wid = lax.axis_index("s") * NC + lax.axis_index("c")
        base = S + wid * RPW
        pltpu.sync_copy(q_hbm, q_v)
        qs = [q_v[pl.ds(16 * j, 16)] for j in range(NV)]

        def start(g, b):
            pltpu.async_copy(
                keys_hbm.at[pl.ds((base + g * CH) * D, CH * D)],
                bufs[b], sems[b])

        def wait(b):
            pltpu.make_async_copy(
                keys_hbm.at[pl.ds(0, CH * D)], bufs[b], sems[b]).wait()

        for b in range(NBUF):
            start(b, b)

        def chunk_body(i, carry, b):
            bv, bi = carry
            g = i * NBUF + b
            wait(b)
            gbase = base + g * CH

            def body(r5, carry):
                bv, bi = carry
                for u in range(U):
                    r = r5 * U + u
                    d = [jnp.abs(bufs[b][pl.ds(r * D + 16 * j, 16)] - qs[j])
                         for j in range(NV)]
                    s1 = [d[0] + d[1], d[2] + d[3], d[4] + d[5], d[6] + d[7]]
                    acc = (s1[0] + s1[1]) + (s1[2] + s1[3])
                    s = jnp.sum(acc)
                    pred = s < bv
                    bv = jnp.where(pred, s, bv)
                    bi = jnp.where(pred, gbase + r, bi)
                return bv, bi

            bv, bi = lax.fori_loop(0, CH // U, body, (bv, bi))

            @pl.when(i < NROUND - 1)
            def _():
                start(g + NBUF, b)

            return bv, bi

        def round_body(i, carry):
            for b in range(NBUF):
                carry = chunk_body(i, carry, b)
            return carry

        bv, bi = lax.fori_loop(
            0, NROUND, round_body, (jnp.float32(jnp.inf), jnp.int32(0)))

        res_v[...] = jnp.full((16,), bv, jnp.float32)
        resi_v[...] = jnp.full((16,), bi, jnp.int32)
        pltpu.sync_copy(res_v, out_v_hbm.at[wid])
        pltpu.sync_copy(resi_v, out_i_hbm.at[wid])

    return k(keys_flat, query_flat)


NACC = 4
HB = BT // 2


def _tc_l1_argmin(keys, query):
    def body(kref0, kref1, qref, ov, oi, best_v, best_i):
        step = pl.program_id(0)

        @pl.when(step == 0)
        def _():
            for a in range(NACC):
                best_v[a, :] = jnp.full((128,), jnp.inf, jnp.float32)
                best_i[a, :] = jnp.zeros((128,), jnp.int32)

        q = qref[...]
        lane = lax.broadcasted_iota(jnp.int32, (1, D), 1)
        bv = [best_v[a, :].reshape(1, D) for a in range(NACC)]
        bi = [best_i[a, :].reshape(1, D) for a in range(NACC)]
        for t in range(TPB):
            kref = kref0 if t < TPB // 2 else kref1
            to = t if t < TPB // 2 else t - TPB // 2
            a = jnp.abs(kref[pl.ds(to * 128, 128), :] - q)
            s = jnp.sum(a.T, axis=0, keepdims=True)
            idx = lane + (step * BT + t * 128)
            c = t % NACC
            pred = s < bv[c]
            bv[c] = jnp.where(pred, s, bv[c])
            bi[c] = jnp.where(pred, idx, bi[c])
        for a in range(NACC):
            best_v[a, :] = bv[a].reshape(D)
            best_i[a, :] = bi[a].reshape(D)

        @pl.when(step == NT - 1)
        def _():
            mv, mi = bv[0], bi[0]
            for a in range(1, NACC):
                p = (bv[a] < mv) | ((bv[a] == mv) & (bi[a] < mi))
                mv = jnp.where(p, bv[a], mv)
                mi = jnp.where(p, bi[a], mi)
            m = jnp.min(mv)
            ci = jnp.min(jnp.where(mv == m, mi, jnp.int32(2 ** 30)))
            ov[0] = m
            oi[0] = ci

    return pl.pallas_call(
        body,
        grid=(NT,),
        in_specs=[pl.BlockSpec((HB, D), lambda i: (2 * i, 0)),
                  pl.BlockSpec((HB, D), lambda i: (2 * i + 1, 0)),
                  pl.BlockSpec((1, D), lambda i: (0, 0))],
        out_specs=[pl.BlockSpec(memory_space=pltpu.SMEM),
                   pl.BlockSpec(memory_space=pltpu.SMEM)],
        out_shape=[jax.ShapeDtypeStruct((1,), jnp.float32),
                   jax.ShapeDtypeStruct((1,), jnp.int32)],
        scratch_shapes=[pltpu.VMEM((NACC, D), jnp.float32),
                        pltpu.VMEM((NACC, D), jnp.int32)],
    )(keys, keys, query)


def kernel(keys, query):
    sc_v, sc_i = _sc_l1_argmin(keys.reshape((K * D,)), query.reshape((D,)))
    tc_v, tc_i = _tc_l1_argmin(keys, query)
    m = jnp.min(sc_v)
    mi = jnp.min(jnp.where(sc_v == m, sc_i, jnp.int32(2 ** 30)))
    pred = (tc_v[0] < m) | ((tc_v[0] == m) & (tc_i[0] < mi))
    return jnp.where(pred, tc_i[0], mi)


# speedup vs baseline: 1.6881x; 1.6881x over previous
import dataclasses
import functools

import jax
import jax.numpy as jnp
from jax import lax
from jax.experimental import pallas as pl
from jax.experimental.pallas import tpu as pltpu
from jax.experimental.pallas import tpu_sc as plsc

K = 100000
D = 128

S = 56320
KSC = K - S

NC = 2
NS = 16
NW = NC * NS
RPW = KSC // NW
NBUF = 5
CH = 91
NROUND = RPW // (NBUF * CH)
assert RPW == NBUF * CH * NROUND
U = 5
NV = D // 16

BT = 2560
NT = S // BT
assert S == BT * NT
TPB = BT // 128


def _sc_l1_argmin(keys_flat, query_flat):
    mesh = plsc.VectorSubcoreMesh(core_axis_name="c", subcore_axis_name="s")
    cp = pltpu.CompilerParams()
    if "needs_layout_passes" in pltpu.CompilerParams.__dataclass_fields__:
        cp = dataclasses.replace(cp, needs_layout_passes=False)

    @functools.partial(
        pl.kernel,
        mesh=mesh,
        compiler_params=cp,
        out_type=[
            jax.ShapeDtypeStruct((NW, 16), jnp.float32),
            jax.ShapeDtypeStruct((NW, 16), jnp.int32),
        ],
        scratch_types=[pltpu.VMEM((CH * D,), jnp.float32)] * NBUF + [
            pltpu.VMEM((D,), jnp.float32),
            pltpu.VMEM((16,), jnp.float32),
            pltpu.VMEM((16,), jnp.int32),
        ] + [pltpu.SemaphoreType.DMA] * NBUF,
    )
    def k(keys_hbm, q_hbm, out_v_hbm, out_i_hbm, *rest):
        bufs = rest[:NBUF]
        q_v, res_v, resi_v = rest[NBUF:NBUF + 3]
        sems = rest[NBUF + 3:]
        wid = lax.axis_index("s") * NC + lax.axis_index("c")
        base = S + wid * RPW
        pltpu.sync_copy(q_hbm, q_v)
        qs = [q_v[pl.ds(16 * j, 16)] for j in range(NV)]

        def start(g, b):
            pltpu.async_copy(
                keys_hbm.at[pl.ds((base + g * CH) * D, CH * D)],
                bufs[b], sems[b])

        def wait(b):
            pltpu.make_async_copy(
                keys_hbm.at[pl.ds(0, CH * D)], bufs[b], sems[b]).wait()

        for b in range(NBUF):
            start(b, b)

        def chunk_body(i, carry, b):
            bv, bi = carry
            g = i * NBUF + b
            wait(b)
            gbase = base + g * CH

            def body(r5, carry):
                bv, bi = carry
                for u in range(U):
                    r = r5 * U + u
                    d = [jnp.abs(bufs[b][pl.ds(r * D + 16 * j, 16)] - qs[j])
                         for j in range(NV)]
                    s1 = [d[0] + d[1], d[2] + d[3], d[4] + d[5], d[6] + d[7]]
                    acc = (s1[0] + s1[1]) + (s1[2] + s1[3])
                    s = jnp.sum(acc)
                    pred = s < bv
                    bv = jnp.where(pred, s, bv)
                    bi = jnp.where(pred, gbase + r, bi)
                return bv, bi

            bv, bi = lax.fori_loop(0, CH // U, body, (bv, bi))

            @pl.when(i < NROUND - 1)
            def _():
                start(g + NBUF, b)

            return bv, bi

        def round_body(i, carry):
            for b in range(NBUF):
                carry = chunk_body(i, carry, b)
            return carry

        bv, bi = lax.fori_loop(
            0, NROUND, round_body, (jnp.float32(jnp.inf), jnp.int32(0)))

        res_v[...] = jnp.full((16,), bv, jnp.float32)
        resi_v[...] = jnp.full((16,), bi, jnp.int32)
        pltpu.sync_copy(res_v, out_v_hbm.at[wid])
        pltpu.sync_copy(resi_v, out_i_hbm.at[wid])

    return k(keys_flat, query_flat)


NACC = 4
NCHT = S // BT
NBT = 2
NRT = NCHT // NBT
assert NCHT == NBT * NRT


def _tc_l1_argmin(keys, query):
    def body(kany, qref, ov, oi, buf0, buf1, sem0, sem1):
        q = qref[...]
        lane = lax.broadcasted_iota(jnp.int32, (1, D), 1)
        bufs = (buf0, buf1)
        sems = (sem0, sem1)

        def start(g, b):
            pltpu.make_async_copy(
                kany.at[pl.ds(g * BT, BT), :], bufs[b], sems[b]).start()

        def wait(b):
            pltpu.make_async_copy(
                kany.at[pl.ds(0, BT), :], bufs[b], sems[b]).wait()

        start(0, 0)
        start(1, 1)

        def chunk(g, carry, b):
            bv, bi = carry
            bv = list(bv)
            bi = list(bi)
            wait(b)
            gbase = g * BT
            for t in range(TPB):
                a = jnp.abs(bufs[b][pl.ds(t * 128, 128), :] - q)
                s = jnp.sum(a.T, axis=0, keepdims=True)
                idx = lane + (gbase + t * 128)
                c = t % NACC
                pred = s < bv[c]
                bv[c] = jnp.where(pred, s, bv[c])
                bi[c] = jnp.where(pred, idx, bi[c])

            @pl.when(g + NBT < NCHT)
            def _():
                start(g + NBT, b)

            return tuple(bv), tuple(bi)

        def round_body(r, carry):
            for b in range(NBT):
                carry = chunk(r * NBT + b, carry, b)
            return carry

        init = (tuple(jnp.full((1, D), jnp.inf, jnp.float32)
                      for _ in range(NACC)),
                tuple(jnp.zeros((1, D), jnp.int32) for _ in range(NACC)))
        bv, bi = lax.fori_loop(0, NRT, round_body, init)

        mv, mi = bv[0], bi[0]
        for a in range(1, NACC):
            p = (bv[a] < mv) | ((bv[a] == mv) & (bi[a] < mi))
            mv = jnp.where(p, bv[a], mv)
            mi = jnp.where(p, bi[a], mi)
        m = jnp.min(mv)
        ci = jnp.min(jnp.where(mv == m, mi, jnp.int32(2 ** 30)))
        ov[0] = m
        oi[0] = ci

    return pl.pallas_call(
        body,
        in_specs=[pl.BlockSpec(memory_space=pl.ANY),
                  pl.BlockSpec(memory_space=pltpu.VMEM)],
        out_specs=[pl.BlockSpec(memory_space=pltpu.SMEM),
                   pl.BlockSpec(memory_space=pltpu.SMEM)],
        out_shape=[jax.ShapeDtypeStruct((1,), jnp.float32),
                   jax.ShapeDtypeStruct((1,), jnp.int32)],
        scratch_shapes=[pltpu.VMEM((BT, D), jnp.float32),
                        pltpu.VMEM((BT, D), jnp.float32),
                        pltpu.SemaphoreType.DMA,
                        pltpu.SemaphoreType.DMA],
    )(keys, query)


def kernel(keys, query):
    sc_v, sc_i = _sc_l1_argmin(keys.reshape((K * D,)), query.reshape((D,)))
    tc_v, tc_i = _tc_l1_argmin(keys, query)
    m = jnp.min(sc_v)
    mi = jnp.min(jnp.where(sc_v == m, sc_i, jnp.int32(2 ** 30)))
    pred = (tc_v[0] < m) | ((tc_v[0] == m) & (tc_i[0] < mi))
    return jnp.where(pred, tc_i[0], mi)


# speedup vs baseline: 1.7942x; 1.0628x over previous
import dataclasses
import functools

import jax
import jax.numpy as jnp
from jax import lax
from jax.experimental import pallas as pl
from jax.experimental.pallas import tpu as pltpu
from jax.experimental.pallas import tpu_sc as plsc

K = 100000
D = 128

S = 56320
KSC = K - S

NC = 2
NS = 16
NW = NC * NS
RPW = KSC // NW
NBUF = 5
CH = 91
NROUND = RPW // (NBUF * CH)
assert RPW == NBUF * CH * NROUND
U = 5
NV = D // 16

BT = 2816
NT = S // BT
assert S == BT * NT
TPB = BT // 128


def _sc_l1_argmin(keys_flat, query_flat):
    mesh = plsc.VectorSubcoreMesh(core_axis_name="c", subcore_axis_name="s")
    cp = pltpu.CompilerParams()
    if "needs_layout_passes" in pltpu.CompilerParams.__dataclass_fields__:
        cp = dataclasses.replace(cp, needs_layout_passes=False)

    @functools.partial(
        pl.kernel,
        mesh=mesh,
        compiler_params=cp,
        out_type=[
            jax.ShapeDtypeStruct((NW, 16), jnp.float32),
            jax.ShapeDtypeStruct((NW, 16), jnp.int32),
        ],
        scratch_types=[pltpu.VMEM((CH * D,), jnp.float32)] * NBUF + [
            pltpu.VMEM((D,), jnp.float32),
            pltpu.VMEM((16,), jnp.float32),
            pltpu.VMEM((16,), jnp.int32),
        ] + [pltpu.SemaphoreType.DMA] * NBUF,
    )
    def k(keys_hbm, q_hbm, out_v_hbm, out_i_hbm, *rest):
        bufs = rest[:NBUF]
        q_v, res_v, resi_v = rest[NBUF:NBUF + 3]
        sems = rest[NBUF + 3:]
        wid = lax.axis_index("s") * NC + lax.axis_index("c")
        base = S + wid * RPW
        pltpu.sync_copy(q_hbm, q_v)
        qs = [q_v[pl.ds(16 * j, 16)] for j in range(NV)]

        def start(g, b):
            pltpu.async_copy(
                keys_hbm.at[pl.ds((base + g * CH) * D, CH * D)],
                bufs[b], sems[b])

        def wait(b):
            pltpu.make_async_copy(
                keys_hbm.at[pl.ds(0, CH * D)], bufs[b], sems[b]).wait()

        for b in range(NBUF):
            start(b, b)

        def chunk_body(i, carry, b):
            bv, bi = carry
            g = i * NBUF + b
            wait(b)
            gbase = base + g * CH

            def body(r5, carry):
                bv, bi = carry
                for u in range(U):
                    r = r5 * U + u
                    d = [jnp.abs(bufs[b][pl.ds(r * D + 16 * j, 16)] - qs[j])
                         for j in range(NV)]
                    s1 = [d[0] + d[1], d[2] + d[3], d[4] + d[5], d[6] + d[7]]
                    acc = (s1[0] + s1[1]) + (s1[2] + s1[3])
                    s = jnp.sum(acc)
                    pred = s < bv
                    bv = jnp.where(pred, s, bv)
                    bi = jnp.where(pred, gbase + r, bi)
                return bv, bi

            bv, bi = lax.fori_loop(0, CH // U, body, (bv, bi))

            @pl.when(i < NROUND - 1)
            def _():
                start(g + NBUF, b)

            return bv, bi

        def round_body(i, carry):
            for b in range(NBUF):
                carry = chunk_body(i, carry, b)
            return carry

        bv, bi = lax.fori_loop(
            0, NROUND, round_body, (jnp.float32(jnp.inf), jnp.int32(0)))

        res_v[...] = jnp.full((16,), bv, jnp.float32)
        resi_v[...] = jnp.full((16,), bi, jnp.int32)
        pltpu.sync_copy(res_v, out_v_hbm.at[wid])
        pltpu.sync_copy(resi_v, out_i_hbm.at[wid])

    return k(keys_flat, query_flat)


NACC = 4
NCHT = S // BT
NBT = 4
NRT = NCHT // NBT
assert NCHT == NBT * NRT


def _tc_l1_argmin(keys, query):
    def body(kany, qref, ov, oi, *rest):
        bufs = rest[:NBT]
        sems = rest[NBT:]
        q = qref[...]
        lane = lax.broadcasted_iota(jnp.int32, (1, D), 1)

        def start(g, b):
            pltpu.make_async_copy(
                kany.at[pl.ds(g * BT, BT), :], bufs[b], sems[b]).start()

        def wait(b):
            pltpu.make_async_copy(
                kany.at[pl.ds(0, BT), :], bufs[b], sems[b]).wait()

        for b in range(NBT):
            start(b, b)

        def chunk(g, carry, b):
            bv, bi = carry
            bv = list(bv)
            bi = list(bi)
            wait(b)
            gbase = g * BT
            for t in range(TPB):
                a = jnp.abs(bufs[b][pl.ds(t * 128, 128), :] - q)
                s = jnp.sum(a.T, axis=0, keepdims=True)
                idx = lane + (gbase + t * 128)
                c = t % NACC
                pred = s < bv[c]
                bv[c] = jnp.where(pred, s, bv[c])
                bi[c] = jnp.where(pred, idx, bi[c])

            @pl.when(g + NBT < NCHT)
            def _():
                start(g + NBT, b)

            return tuple(bv), tuple(bi)

        def round_body(r, carry):
            for b in range(NBT):
                carry = chunk(r * NBT + b, carry, b)
            return carry

        init = (tuple(jnp.full((1, D), jnp.inf, jnp.float32)
                      for _ in range(NACC)),
                tuple(jnp.zeros((1, D), jnp.int32) for _ in range(NACC)))
        bv, bi = lax.fori_loop(0, NRT, round_body, init)

        mv, mi = bv[0], bi[0]
        for a in range(1, NACC):
            p = (bv[a] < mv) | ((bv[a] == mv) & (bi[a] < mi))
            mv = jnp.where(p, bv[a], mv)
            mi = jnp.where(p, bi[a], mi)
        m = jnp.min(mv)
        ci = jnp.min(jnp.where(mv == m, mi, jnp.int32(2 ** 30)))
        ov[0] = m
        oi[0] = ci

    return pl.pallas_call(
        body,
        in_specs=[pl.BlockSpec(memory_space=pl.ANY),
                  pl.BlockSpec(memory_space=pltpu.VMEM)],
        out_specs=[pl.BlockSpec(memory_space=pltpu.SMEM),
                   pl.BlockSpec(memory_space=pltpu.SMEM)],
        out_shape=[jax.ShapeDtypeStruct((1,), jnp.float32),
                   jax.ShapeDtypeStruct((1,), jnp.int32)],
        scratch_shapes=[pltpu.VMEM((BT, D), jnp.float32)] * NBT +
                       [pltpu.SemaphoreType.DMA] * NBT,
    )(keys, query)


def kernel(keys, query):
    sc_v, sc_i = _sc_l1_argmin(keys.reshape((K * D,)), query.reshape((D,)))
    tc_v, tc_i = _tc_l1_argmin(keys, query)
    m = jnp.min(sc_v)
    mi = jnp.min(jnp.where(sc_v == m, sc_i, jnp.int32(2 ** 30)))
    pred = (tc_v[0] < m) | ((tc_v[0] == m) & (tc_i[0] < mi))
    return jnp.where(pred, tc_i[0], mi)


# speedup vs baseline: 1.8421x; 1.0267x over previous
import dataclasses
import functools

import jax
import jax.numpy as jnp
from jax import lax
from jax.experimental import pallas as pl
from jax.experimental.pallas import tpu as pltpu
from jax.experimental.pallas import tpu_sc as plsc

K = 100000
D = 128

S = 64000
KSC = K - S

NC = 2
NS = 16
NW = NC * NS
RPW = KSC // NW
NBUF = 5
CH = 75
NROUND = RPW // (NBUF * CH)
assert RPW == NBUF * CH * NROUND
U = 5
NV = D // 16

BT = 2560
NT = S // BT
assert S == BT * NT
TPB = BT // 128


def _sc_l1_argmin(keys_flat, query_flat):
    mesh = plsc.VectorSubcoreMesh(core_axis_name="c", subcore_axis_name="s")
    cp = pltpu.CompilerParams()
    if "needs_layout_passes" in pltpu.CompilerParams.__dataclass_fields__:
        cp = dataclasses.replace(cp, needs_layout_passes=False)

    @functools.partial(
        pl.kernel,
        mesh=mesh,
        compiler_params=cp,
        out_type=[
            jax.ShapeDtypeStruct((NW, 16), jnp.float32),
            jax.ShapeDtypeStruct((NW, 16), jnp.int32),
        ],
        scratch_types=[pltpu.VMEM((CH * D,), jnp.float32)] * NBUF + [
            pltpu.VMEM((D,), jnp.float32),
            pltpu.VMEM((16,), jnp.float32),
            pltpu.VMEM((16,), jnp.int32),
        ] + [pltpu.SemaphoreType.DMA] * NBUF,
    )
    def k(keys_hbm, q_hbm, out_v_hbm, out_i_hbm, *rest):
        bufs = rest[:NBUF]
        q_v, res_v, resi_v = rest[NBUF:NBUF + 3]
        sems = rest[NBUF + 3:]
        wid = lax.axis_index("s") * NC + lax.axis_index("c")
        base = S + wid * RPW
        pltpu.sync_copy(q_hbm, q_v)
        qs = [q_v[pl.ds(16 * j, 16)] for j in range(NV)]

        def start(g, b):
            pltpu.async_copy(
                keys_hbm.at[pl.ds((base + g * CH) * D, CH * D)],
                bufs[b], sems[b])

        def wait(b):
            pltpu.make_async_copy(
                keys_hbm.at[pl.ds(0, CH * D)], bufs[b], sems[b]).wait()

        for b in range(NBUF):
            start(b, b)

        def chunk_body(i, carry, b):
            bv, bi = carry
            g = i * NBUF + b
            wait(b)
            gbase = base + g * CH

            def body(r5, carry):
                bv, bi = carry
                for u in range(U):
                    r = r5 * U + u
                    d = [jnp.abs(bufs[b][pl.ds(r * D + 16 * j, 16)] - qs[j])
                         for j in range(NV)]
                    s1 = [d[0] + d[1], d[2] + d[3], d[4] + d[5], d[6] + d[7]]
                    acc = (s1[0] + s1[1]) + (s1[2] + s1[3])
                    s = jnp.sum(acc)
                    pred = s < bv
                    bv = jnp.where(pred, s, bv)
                    bi = jnp.where(pred, gbase + r, bi)
                return bv, bi

            bv, bi = lax.fori_loop(0, CH // U, body, (bv, bi))

            @pl.when(i < NROUND - 1)
            def _():
                start(g + NBUF, b)

            return bv, bi

        def round_body(i, carry):
            for b in range(NBUF):
                carry = chunk_body(i, carry, b)
            return carry

        bv, bi = lax.fori_loop(
            0, NROUND, round_body, (jnp.float32(jnp.inf), jnp.int32(0)))

        res_v[...] = jnp.full((16,), bv, jnp.float32)
        resi_v[...] = jnp.full((16,), bi, jnp.int32)
        pltpu.sync_copy(res_v, out_v_hbm.at[wid])
        pltpu.sync_copy(resi_v, out_i_hbm.at[wid])

    return k(keys_flat, query_flat)


NACC = 4
NCHT = S // BT
NBT = 5
NRT = NCHT // NBT
assert NCHT == NBT * NRT


def _tc_l1_argmin(keys, query):
    def body(kany, qref, ov, oi, *rest):
        bufs = rest[:NBT]
        sems = rest[NBT:]
        q = qref[...]
        lane = lax.broadcasted_iota(jnp.int32, (1, D), 1)

        def start(g, b):
            pltpu.make_async_copy(
                kany.at[pl.ds(g * BT, BT), :], bufs[b], sems[b]).start()

        def wait(b):
            pltpu.make_async_copy(
                kany.at[pl.ds(0, BT), :], bufs[b], sems[b]).wait()

        for b in range(NBT):
            start(b, b)

        def chunk(g, carry, b):
            bv, bi = carry
            bv = list(bv)
            bi = list(bi)
            wait(b)
            gbase = g * BT
            for t in range(TPB):
                a = jnp.abs(bufs[b][pl.ds(t * 128, 128), :] - q)
                s = jnp.sum(a.T, axis=0, keepdims=True)
                idx = lane + (gbase + t * 128)
                c = t % NACC
                pred = s < bv[c]
                bv[c] = jnp.where(pred, s, bv[c])
                bi[c] = jnp.where(pred, idx, bi[c])

            @pl.when(g + NBT < NCHT)
            def _():
                start(g + NBT, b)

            return tuple(bv), tuple(bi)

        def round_body(r, carry):
            for b in range(NBT):
                carry = chunk(r * NBT + b, carry, b)
            return carry

        init = (tuple(jnp.full((1, D), jnp.inf, jnp.float32)
                      for _ in range(NACC)),
                tuple(jnp.zeros((1, D), jnp.int32) for _ in range(NACC)))
        bv, bi = lax.fori_loop(0, NRT, round_body, init)

        mv, mi = bv[0], bi[0]
        for a in range(1, NACC):
            p = (bv[a] < mv) | ((bv[a] == mv) & (bi[a] < mi))
            mv = jnp.where(p, bv[a], mv)
            mi = jnp.where(p, bi[a], mi)
        m = jnp.min(mv)
        ci = jnp.min(jnp.where(mv == m, mi, jnp.int32(2 ** 30)))
        ov[0] = m
        oi[0] = ci

    return pl.pallas_call(
        body,
        in_specs=[pl.BlockSpec(memory_space=pl.ANY),
                  pl.BlockSpec(memory_space=pltpu.VMEM)],
        out_specs=[pl.BlockSpec(memory_space=pltpu.SMEM),
                   pl.BlockSpec(memory_space=pltpu.SMEM)],
        out_shape=[jax.ShapeDtypeStruct((1,), jnp.float32),
                   jax.ShapeDtypeStruct((1,), jnp.int32)],
        scratch_shapes=[pltpu.VMEM((BT, D), jnp.float32)] * NBT +
                       [pltpu.SemaphoreType.DMA] * NBT,
    )(keys, query)


def kernel(keys, query):
    sc_v, sc_i = _sc_l1_argmin(keys.reshape((K * D,)), query.reshape((D,)))
    tc_v, tc_i = _tc_l1_argmin(keys, query)
    m = jnp.min(sc_v)
    mi = jnp.min(jnp.where(sc_v == m, sc_i, jnp.int32(2 ** 30)))
    pred = (tc_v[0] < m) | ((tc_v[0] == m) & (tc_i[0] < mi))
    return jnp.where(pred, tc_i[0], mi)


# speedup vs baseline: 2.1265x; 1.1543x over previous
import dataclasses
import functools

import jax
import jax.numpy as jnp
from jax import lax
from jax.experimental import pallas as pl
from jax.experimental.pallas import tpu as pltpu
from jax.experimental.pallas import tpu_sc as plsc

K = 100000
D = 128

S = 64000
KSC = K - S

NC = 2
NS = 16
NW = NC * NS
RPW = KSC // NW
NBUF = 5
CH = 75
NROUND = RPW // (NBUF * CH)
assert RPW == NBUF * CH * NROUND
U = 5
NV = D // 16

BT = 2560
NT = S // BT
assert S == BT * NT
TPB = BT // 128


def _sc_l1_argmin(keys_flat, query_flat):
    mesh = plsc.VectorSubcoreMesh(core_axis_name="c", subcore_axis_name="s")
    cp = pltpu.CompilerParams()
    if "needs_layout_passes" in pltpu.CompilerParams.__dataclass_fields__:
        cp = dataclasses.replace(cp, needs_layout_passes=False)

    @functools.partial(
        pl.kernel,
        mesh=mesh,
        compiler_params=cp,
        out_type=[
            jax.ShapeDtypeStruct((NW, 16), jnp.float32),
            jax.ShapeDtypeStruct((NW, 16), jnp.int32),
        ],
        scratch_types=[pltpu.VMEM((CH * D,), jnp.float32)] * NBUF + [
            pltpu.VMEM((D,), jnp.float32),
            pltpu.VMEM((16,), jnp.float32),
            pltpu.VMEM((16,), jnp.int32),
        ] + [pltpu.SemaphoreType.DMA] * NBUF,
    )
    def k(keys_hbm, q_hbm, out_v_hbm, out_i_hbm, *rest):
        bufs = rest[:NBUF]
        q_v, res_v, resi_v = rest[NBUF:NBUF + 3]
        sems = rest[NBUF + 3:]
        wid = lax.axis_index("s") * NC + lax.axis_index("c")
        base = S + wid * RPW
        pltpu.sync_copy(q_hbm, q_v)
        qs = [q_v[pl.ds(16 * j, 16)] for j in range(NV)]

        def start(g, b):
            pltpu.async_copy(
                keys_hbm.at[pl.ds((base + g * CH) * D, CH * D)],
                bufs[b], sems[b])

        def wait(b):
            pltpu.make_async_copy(
                keys_hbm.at[pl.ds(0, CH * D)], bufs[b], sems[b]).wait()

        for b in range(NBUF):
            start(b, b)

        def chunk_body(i, carry, b):
            bv, bi = carry
            g = i * NBUF + b
            wait(b)
            gbase = base + g * CH

            def body(r5, carry):
                bv, bi = carry
                for u in range(U):
                    r = r5 * U + u
                    d = [jnp.abs(bufs[b][pl.ds(r * D + 16 * j, 16)] - qs[j])
                         for j in range(NV)]
                    s1 = [d[0] + d[1], d[2] + d[3], d[4] + d[5], d[6] + d[7]]
                    acc = (s1[0] + s1[1]) + (s1[2] + s1[3])
                    s = jnp.sum(acc)
                    pred = s < bv
                    bv = jnp.where(pred, s, bv)
                    bi = jnp.where(pred, gbase + r, bi)
                return bv, bi

            bv, bi = lax.fori_loop(0, CH // U, body, (bv, bi))

            @pl.when(i < NROUND - 1)
            def _():
                start(g + NBUF, b)

            return bv, bi

        def round_body(i, carry):
            for b in range(NBUF):
                carry = chunk_body(i, carry, b)
            return carry

        bv, bi = lax.fori_loop(
            0, NROUND, round_body, (jnp.float32(jnp.inf), jnp.int32(0)))

        res_v[...] = jnp.full((16,), bv, jnp.float32)
        resi_v[...] = jnp.full((16,), bi, jnp.int32)
        pltpu.sync_copy(res_v, out_v_hbm.at[wid])
        pltpu.sync_copy(resi_v, out_i_hbm.at[wid])

    return k(keys_flat, query_flat)


NACC = 4
NCHT = S // BT
NBT = 5
NRT = NCHT // NBT
assert NCHT == NBT * NRT


def _tc_l1_argmin(keys, query):
    def body(kany, qref, ov, oi, *rest):
        bufs = rest[:NBT]
        sems = rest[NBT:]
        q = qref[...]
        lane = lax.broadcasted_iota(jnp.int32, (1, D), 1)

        def start(g, b):
            pltpu.make_async_copy(
                kany.at[pl.ds(g * BT, BT), :], bufs[b], sems[b]).start()

        def wait(b):
            pltpu.make_async_copy(
                kany.at[pl.ds(0, BT), :], bufs[b], sems[b]).wait()

        for b in range(NBT):
            start(b, b)

        def chunk(g, carry, b):
            bv, bi = carry
            bv = list(bv)
            bi = list(bi)
            wait(b)
            gbase = g * BT
            for t in range(TPB):
                a = jnp.abs(bufs[b][pl.ds(t * 128, 128), :] - q)
                s = jnp.sum(a.T, axis=0, keepdims=True)
                idx = lane + (gbase + t * 128)
                c = t % NACC
                pred = s < bv[c]
                bv[c] = jnp.where(pred, s, bv[c])
                bi[c] = jnp.where(pred, idx, bi[c])

            @pl.when(g + NBT < NCHT)
            def _():
                start(g + NBT, b)

            return tuple(bv), tuple(bi)

        def round_body(r, carry):
            for b in range(NBT):
                carry = chunk(r * NBT + b, carry, b)
            return carry

        init = (tuple(jnp.full((1, D), jnp.inf, jnp.float32)
                      for _ in range(NACC)),
                tuple(jnp.zeros((1, D), jnp.int32) for _ in range(NACC)))
        bv, bi = lax.fori_loop(0, NRT, round_body, init)

        mv, mi = bv[0], bi[0]
        for a in range(1, NACC):
            p = (bv[a] < mv) | ((bv[a] == mv) & (bi[a] < mi))
            mv = jnp.where(p, bv[a], mv)
            mi = jnp.where(p, bi[a], mi)
        m = jnp.min(mv)
        ci = jnp.min(jnp.where(mv == m, mi, jnp.int32(2 ** 30)))
        ov[0] = m
        oi[0] = ci

    return pl.pallas_call(
        body,
        in_specs=[pl.BlockSpec(memory_space=pl.ANY),
                  pl.BlockSpec(memory_space=pltpu.VMEM)],
        out_specs=[pl.BlockSpec(memory_space=pltpu.SMEM),
                   pl.BlockSpec(memory_space=pltpu.SMEM)],
        out_shape=[jax.ShapeDtypeStruct((1,), jnp.float32),
                   jax.ShapeDtypeStruct((1,), jnp.int32)],
        scratch_shapes=[pltpu.VMEM((BT, D), jnp.float32)] * NBT +
                       [pltpu.SemaphoreType.DMA] * NBT,
    )(keys, query)


def _combine(sc_v, sc_i, tc_v, tc_i):
    def body(svr, sir, tvr, tir, o):
        sv = svr[...]
        si = sir[...]
        m = jnp.min(sv)
        mi = jnp.min(jnp.where(sv == m, si, jnp.int32(2 ** 30)))
        tv = tvr[0]
        ti = tir[0]
        pred = (tv < m) | ((tv == m) & (ti < mi))
        o[0] = jnp.where(pred, ti, mi)

    return pl.pallas_call(
        body,
        in_specs=[pl.BlockSpec(memory_space=pltpu.VMEM),
                  pl.BlockSpec(memory_space=pltpu.VMEM),
                  pl.BlockSpec(memory_space=pltpu.SMEM),
                  pl.BlockSpec(memory_space=pltpu.SMEM)],
        out_specs=pl.BlockSpec(memory_space=pltpu.SMEM),
        out_shape=jax.ShapeDtypeStruct((1,), jnp.int32),
    )(sc_v, sc_i, tc_v, tc_i)


def kernel(keys, query):
    sc_v, sc_i = _sc_l1_argmin(keys.reshape((K * D,)), query.reshape((D,)))
    tc_v, tc_i = _tc_l1_argmin(keys, query)
    return _combine(sc_v, sc_i, tc_v, tc_i).reshape(())
